# Initial kernel scaffold; baseline (speedup 1.0000x reference)
import jax
import jax.numpy as jnp
from jax.experimental import pallas as pl


def kernel(token_ids, token_table, pos_table):
    raise NotImplementedError("write your pallas kernel here")


# speedup vs baseline: 1.3778x; 1.3778x over previous
import functools

import jax
import jax.numpy as jnp
from jax import lax
from jax.experimental import pallas as pl
from jax.experimental.pallas import tpu as pltpu
from jax.experimental.pallas import tpu_sc as plsc

VOCAB = 100000
MAX_SEQ = 8192
D_MODEL = 1024
BATCH = 4
SEQ = 8192

_INFO = plsc.get_sparse_core_info()
_NC = _INFO.num_cores
_NS = _INFO.num_subcores
_NW = _NC * _NS
_LANES = _INFO.num_lanes

POS_PER_W = SEQ // _NW
CHUNK = 32
N_CHUNKS = POS_PER_W // CHUNK


def _make_kernel():
    mesh = plsc.VectorSubcoreMesh(core_axis_name="c", subcore_axis_name="s")

    @functools.partial(
        pl.kernel,
        mesh=mesh,
        out_type=jax.ShapeDtypeStruct((BATCH * SEQ, D_MODEL), jnp.float32),
        scratch_types=[
            pltpu.VMEM((CHUNK,), jnp.int32),
            pltpu.VMEM((CHUNK, D_MODEL), jnp.float32),
            pltpu.VMEM((CHUNK, D_MODEL), jnp.float32),
            pltpu.SemaphoreType.DMA,
        ],
    )
    def emb_kernel(ids_hbm, tok_hbm, pos_hbm, out_hbm, idx_v, rows_v, pos_v, sem):
        wid = lax.axis_index("s") * _NC + lax.axis_index("c")
        pos_base = wid * POS_PER_W

        def chunk_body(c, _):
            pos_off = pos_base + c * CHUNK
            pltpu.sync_copy(pos_hbm.at[pl.ds(pos_off, CHUNK)], pos_v)

            def batch_body(b, _):
                row_base = b * SEQ + pos_off
                pltpu.sync_copy(ids_hbm.at[pl.ds(row_base, CHUNK)], idx_v)
                pltpu.async_copy(tok_hbm.at[idx_v], rows_v, sem).wait()

                def add_row(r, _):
                    for j in range(D_MODEL // _LANES):
                        sl = pl.ds(j * _LANES, _LANES)
                        rows_v[r, sl] = rows_v[r, sl] + pos_v[r, sl]
                    return 0

                lax.fori_loop(0, CHUNK, add_row, 0)
                pltpu.sync_copy(rows_v, out_hbm.at[pl.ds(row_base, CHUNK)])
                return 0

            lax.fori_loop(0, BATCH, batch_body, 0)
            return 0

        lax.fori_loop(0, N_CHUNKS, chunk_body, 0)

    return emb_kernel


_EMB_KERNEL = _make_kernel()


@jax.jit
def kernel(token_ids, token_table, pos_table):
    ids_flat = token_ids.reshape(BATCH * SEQ).astype(jnp.int32)
    out = _EMB_KERNEL(ids_flat, token_table, pos_table)
    return out.reshape(BATCH, SEQ, D_MODEL)


# speedup vs baseline: 2.4634x; 1.7879x over previous
import functools

import jax
import jax.numpy as jnp
from jax import lax
from jax.experimental import pallas as pl
from jax.experimental.pallas import tpu as pltpu
from jax.experimental.pallas import tpu_sc as plsc

VOCAB = 100000
MAX_SEQ = 8192
D_MODEL = 1024
BATCH = 4
SEQ = 8192

_INFO = plsc.get_sparse_core_info()
_NC = _INFO.num_cores
_NS = _INFO.num_subcores
_NW = _NC * _NS
_LANES = _INFO.num_lanes

POS_PER_W = SEQ // _NW
CHUNK = 16
N_CHUNKS = POS_PER_W // CHUNK


def _make_kernel():
    mesh = plsc.VectorSubcoreMesh(core_axis_name="c", subcore_axis_name="s")

    @functools.partial(
        pl.kernel,
        mesh=mesh,
        out_type=jax.ShapeDtypeStruct((BATCH * SEQ, D_MODEL), jnp.float32),
        scratch_types=[
            pltpu.VMEM((BATCH, POS_PER_W), jnp.int32),
            pltpu.VMEM((CHUNK, D_MODEL), jnp.float32),
            pltpu.VMEM((CHUNK, D_MODEL), jnp.float32),
            pltpu.VMEM((CHUNK, D_MODEL), jnp.float32),
            pltpu.VMEM((CHUNK, D_MODEL), jnp.float32),
            pltpu.VMEM((CHUNK, D_MODEL), jnp.float32),
            pltpu.VMEM((CHUNK, D_MODEL), jnp.float32),
            pltpu.SemaphoreType.DMA,
            pltpu.SemaphoreType.DMA,
            pltpu.SemaphoreType.DMA,
            pltpu.SemaphoreType.DMA,
            pltpu.SemaphoreType.DMA,
            pltpu.SemaphoreType.DMA,
        ],
    )
    def emb_kernel(ids_hbm, tok_hbm, pos_hbm, out_hbm,
                   idxv, rows0, rows1, stg0, stg1, pos0, pos1,
                   gsem0, gsem1, ssem0, ssem1, psem0, psem1):
        wid = lax.axis_index("s") * _NC + lax.axis_index("c")
        pos_base = wid * POS_PER_W
        rows = (rows0, rows1)
        stg = (stg0, stg1)
        posb = (pos0, pos1)
        gsem = (gsem0, gsem1)
        ssem = (ssem0, ssem1)
        psem = (psem0, psem1)

        def idx_view(c, b):
            return idxv.at[b, pl.ds(c * CHUNK, CHUNK)]

        def out_view(c, b):
            return out_hbm.at[pl.ds(b * SEQ + pos_base + c * CHUNK, CHUNK)]

        def pos_view(c):
            return pos_hbm.at[pl.ds(pos_base + c * CHUNK, CHUNK)]

        def start_gather(c, b, kb):
            pltpu.async_copy(tok_hbm.at[idx_view(c, b)], rows[kb], gsem[kb])

        def wait_gather(kb):
            pltpu.make_async_copy(tok_hbm.at[idx_view(0, 0)], rows[kb],
                                  gsem[kb]).wait()

        def start_store(c, b, kb):
            pltpu.async_copy(stg[kb], out_view(c, b), ssem[kb])

        def wait_store(kb):
            pltpu.make_async_copy(stg[kb], out_view(0, 0), ssem[kb]).wait()

        def start_pos(c, pb):
            pltpu.async_copy(pos_view(c), posb[pb], psem[pb])

        def wait_pos(pb):
            pltpu.make_async_copy(pos_view(0), posb[pb], psem[pb]).wait()

        def add_chunk(kb, pb):
            def row(r, _):
                for j in range(D_MODEL // _LANES):
                    sl = pl.ds(j * _LANES, _LANES)
                    stg[kb][r, sl] = rows[kb][r, sl] + posb[pb][r, sl]
                return 0

            lax.fori_loop(0, CHUNK, row, 0)

        for b in range(BATCH):
            pltpu.sync_copy(ids_hbm.at[b, pl.ds(pos_base, POS_PER_W)],
                            idxv.at[b])
        start_pos(0, 0)
        start_gather(0, 0, 0)
        start_gather(0, 1, 1)

        def group(gg, _):
            for dg in range(2):
                g = gg * 2 + dg
                for k in range(BATCH):
                    kb = k % 2
                    if k == 0:
                        wait_pos(dg)
                        if dg == 0:
                            start_pos(g + 1, 1)
                        else:
                            @pl.when(gg < N_CHUNKS // 2 - 1)
                            def _():
                                start_pos(g + 1, 0)
                    wait_gather(kb)
                    if dg == 0 and k < 2:
                        @pl.when(gg > 0)
                        def _():
                            wait_store(kb)
                    else:
                        wait_store(kb)
                    add_chunk(kb, dg)
                    if k < 2:
                        start_gather(g, k + 2, kb)
                    elif dg == 0:
                        start_gather(g + 1, k - 2, kb)
                    else:
                        @pl.when(gg < N_CHUNKS // 2 - 1)
                        def _():
                            start_gather(g + 1, k - 2, kb)
                    start_store(g, k, kb)
            return 0

        lax.fori_loop(0, N_CHUNKS // 2, group, 0)
        wait_store(0)
        wait_store(1)

    return emb_kernel


_EMB_KERNEL = _make_kernel()


@jax.jit
def kernel(token_ids, token_table, pos_table):
    ids = token_ids.astype(jnp.int32)
    out = _EMB_KERNEL(ids, token_table, pos_table)
    return out.reshape(BATCH, SEQ, D_MODEL)
